# baseline (speedup 1.0000x reference)
import functools

import jax
import jax.numpy as jnp
from jax import lax
from jax.experimental import pallas as pl
from jax.experimental.pallas import tpu as pltpu
from jax.experimental.pallas import tpu_sc as plsc

_B = 4096
_L = 200
_D = 32
_V = 1000000
_N = _B * _L
_NW = 32
_IPW = _N // _NW
_C = 1280
_NCH = _IPW // _C
_NOUT = _NCH // 2

_T1_TOK = 2048
_T1_GRID = -(-_V // _T1_TOK)
_VP = _T1_GRID * _T1_TOK


def _t1_body(x_ref, y_ref):
    x = x_ref[...]
    parts = [x[:, 512 * j:512 * (j + 1)].T for j in range(4)]
    y_ref[...] = jnp.concatenate(parts, axis=1)


_t1 = pl.pallas_call(
    _t1_body,
    grid=(_T1_GRID,),
    in_specs=[pl.BlockSpec((_D, _T1_TOK), lambda g: (0, g))],
    out_specs=pl.BlockSpec((_T1_TOK // 4, 128), lambda g: (g, 0)),
    out_shape=jax.ShapeDtypeStruct((_VP * _D // 128, 128), jnp.float32),
)


def _t3_body(x_ref, y_ref):
    y_ref[...] = x_ref[...].T


_t3 = pl.pallas_call(
    _t3_body,
    grid=(8, 10),
    in_specs=[pl.BlockSpec((512, 640), lambda i, j: (i, j))],
    out_specs=pl.BlockSpec((640, 512), lambda i, j: (j, i)),
    out_shape=jax.ShapeDtypeStruct((_L * _D, _B), jnp.float32),
)


def _make_emb_kernel():
    mesh = plsc.VectorSubcoreMesh(core_axis_name="c", subcore_axis_name="s")

    @functools.partial(
        pl.kernel,
        mesh=mesh,
        out_type=jax.ShapeDtypeStruct((_N, _D), jnp.float32),
        scratch_types=[
            pltpu.VMEM((_IPW,), jnp.int32),
            pltpu.VMEM((_C, _D), jnp.float32),
            pltpu.VMEM((_C, _D), jnp.float32),
            pltpu.SemaphoreType.DMA,
            pltpu.SemaphoreType.DMA,
            pltpu.SemaphoreType.DMA,
            pltpu.SemaphoreType.DMA,
        ],
        compiler_params=pltpu.CompilerParams(use_tc_tiling_on_sc=False),
    )
    def emb(idx_hbm, table_hbm, out_hbm, idx_v, rows0, rows1,
            gsem0, gsem1, ssem0, ssem1):
        wid = lax.axis_index("c") * 16 + lax.axis_index("s")
        base = wid * _IPW
        pltpu.sync_copy(idx_hbm.at[pl.ds(base, _IPW)], idx_v)

        def xform(i, carry):
            v = idx_v[pl.ds(i * 16, 16)]
            idx_v[pl.ds(i * 16, 16)] = (
                (v & -2048) | ((v & 511) << 2) | ((v & 2047) >> 9))
            return carry

        lax.fori_loop(0, _IPW // 16, xform, 0)

        rows = (rows0, rows1)
        gsem = (gsem0, gsem1)
        ssem = (ssem0, ssem1)

        def fire(c, b):
            pltpu.make_async_copy(
                table_hbm.at[idx_v.at[pl.ds(c * _C, _C)]],
                rows[b], gsem[b]).start()

        def drain_gather(b):
            pltpu.make_async_copy(
                table_hbm.at[idx_v.at[pl.ds(0, _C)]],
                rows[b], gsem[b]).wait()

        def store_start(c, b):
            pltpu.make_async_copy(
                rows[b], out_hbm.at[pl.ds(base + c * _C, _C)],
                ssem[b]).start()

        def store_wait(b):
            pltpu.make_async_copy(
                rows[b], out_hbm.at[pl.ds(base, _C)], ssem[b]).wait()

        fire(0, 0)
        fire(1, 1)

        def body(t, carry):
            for b in range(2):
                c = 2 * t + b

                drain_gather(b)
                store_start(c, b)

                @pl.when(t < _NOUT - 1)
                def _():
                    store_wait(b)
                    fire(c + 2, b)

            return carry

        lax.fori_loop(0, _NOUT, body, 0)
        store_wait(0)
        store_wait(1)

    return emb


_emb = _make_emb_kernel()


@jax.jit
def kernel(token_id_flat, lengths, table):
    idx = token_id_flat.astype(jnp.int32)
    tbl_rm = _t1(table.T).reshape(_VP, _D)
    out = _emb(idx, tbl_rm)
    o_t = _t3(out.reshape(_B, _L * _D))
    return jnp.transpose(o_t.reshape(_L, _D, _B), (2, 0, 1))


# speedup vs baseline: 1.1926x; 1.1926x over previous
import functools

import jax
import jax.numpy as jnp
from jax import lax
from jax.experimental import pallas as pl
from jax.experimental.pallas import tpu as pltpu
from jax.experimental.pallas import tpu_sc as plsc

_B = 4096
_L = 200
_D = 32
_V = 1000000
_N = _B * _L
_NW = 32
_IPW = _N // _NW
_C = 1280
_NCH = _IPW // _C
_NOUT = _NCH // 2

_T1_TOK = 2048
_T1_GRID = -(-_V // _T1_TOK)
_VP = _T1_GRID * _T1_TOK


def _t1_body(x_ref, y_ref):
    x = x_ref[...]
    parts = [x[:, 512 * j:512 * (j + 1)].T for j in range(4)]
    y_ref[...] = jnp.concatenate(parts, axis=1)


_t1 = pl.pallas_call(
    _t1_body,
    grid=(_T1_GRID,),
    in_specs=[pl.BlockSpec((_D, _T1_TOK), lambda g: (0, g))],
    out_specs=pl.BlockSpec((_T1_TOK // 4, 128), lambda g: (g, 0)),
    out_shape=jax.ShapeDtypeStruct((_VP * _D // 128, 128), jnp.float32),
)


def _t3_body(x_ref, y_ref):
    x = x_ref[...].reshape(256, 50, 128)
    parts = [x[:, m, :].T for m in range(50)]
    y_ref[...] = jnp.concatenate(parts, axis=0)


_t3 = pl.pallas_call(
    _t3_body,
    grid=(16,),
    in_specs=[pl.BlockSpec((12800, 128), lambda i: (i, 0))],
    out_specs=pl.BlockSpec((_L * _D, 256), lambda i: (0, i)),
    out_shape=jax.ShapeDtypeStruct((_L * _D, _B), jnp.float32),
)


def _make_emb_kernel():
    mesh = plsc.VectorSubcoreMesh(core_axis_name="c", subcore_axis_name="s")

    @functools.partial(
        pl.kernel,
        mesh=mesh,
        out_type=jax.ShapeDtypeStruct((_N, _D), jnp.float32),
        scratch_types=[
            pltpu.VMEM((_IPW,), jnp.int32),
            pltpu.VMEM((_C, _D), jnp.float32),
            pltpu.VMEM((_C, _D), jnp.float32),
            pltpu.SemaphoreType.DMA,
            pltpu.SemaphoreType.DMA,
            pltpu.SemaphoreType.DMA,
            pltpu.SemaphoreType.DMA,
        ],
        compiler_params=pltpu.CompilerParams(use_tc_tiling_on_sc=False),
    )
    def emb(idx_hbm, table_hbm, out_hbm, idx_v, rows0, rows1,
            gsem0, gsem1, ssem0, ssem1):
        wid = lax.axis_index("c") * 16 + lax.axis_index("s")
        base = wid * _IPW
        pltpu.sync_copy(idx_hbm.at[pl.ds(base, _IPW)], idx_v)

        def xform(i, carry):
            v = idx_v[pl.ds(i * 16, 16)]
            idx_v[pl.ds(i * 16, 16)] = (
                (v & -2048) | ((v & 511) << 2) | ((v & 2047) >> 9))
            return carry

        lax.fori_loop(0, _IPW // 16, xform, 0)

        rows = (rows0, rows1)
        gsem = (gsem0, gsem1)
        ssem = (ssem0, ssem1)

        def fire(c, b):
            pltpu.make_async_copy(
                table_hbm.at[idx_v.at[pl.ds(c * _C, _C)]],
                rows[b], gsem[b]).start()

        def drain_gather(b):
            pltpu.make_async_copy(
                table_hbm.at[idx_v.at[pl.ds(0, _C)]],
                rows[b], gsem[b]).wait()

        def store_start(c, b):
            pltpu.make_async_copy(
                rows[b], out_hbm.at[pl.ds(base + c * _C, _C)],
                ssem[b]).start()

        def store_wait(b):
            pltpu.make_async_copy(
                rows[b], out_hbm.at[pl.ds(base, _C)], ssem[b]).wait()

        fire(0, 0)
        fire(1, 1)

        def body(t, carry):
            for b in range(2):
                c = 2 * t + b

                drain_gather(b)
                store_start(c, b)

                @pl.when(t < _NOUT - 1)
                def _():
                    store_wait(b)
                    fire(c + 2, b)

            return carry

        lax.fori_loop(0, _NOUT, body, 0)
        store_wait(0)
        store_wait(1)

    return emb


_emb = _make_emb_kernel()


@jax.jit
def kernel(token_id_flat, lengths, table):
    idx = token_id_flat.astype(jnp.int32)
    tbl_rm = _t1(table.T).reshape(_VP, _D)
    out = _emb(idx, tbl_rm)
    o_t = _t3(out.reshape(_N * _D // 128, 128))
    return jnp.transpose(o_t.reshape(_L, _D, _B), (2, 0, 1))


# speedup vs baseline: 1.6552x; 1.3880x over previous
import functools

import jax
import jax.numpy as jnp
from jax import lax
from jax.experimental import pallas as pl
from jax.experimental.pallas import tpu as pltpu
from jax.experimental.pallas import tpu_sc as plsc

_B = 4096
_L = 200
_D = 32
_V = 1000000
_N = _B * _L
_NW = 32
_IPW = _N // _NW
_C = 1280
_NCH = _IPW // _C
_NOUT = _NCH // 2

_T1_TOK = 16384
_T1_GRID = -(-_V // _T1_TOK)
_VP = _T1_GRID * _T1_TOK


def _t1_body(x_ref, y_ref):
    x = x_ref[...]
    groups = []
    for k in range(8):
        parts = [x[:, 2048 * k + 512 * j:2048 * k + 512 * (j + 1)].T
                 for j in range(4)]
        groups.append(jnp.concatenate(parts, axis=1))
    y_ref[...] = jnp.concatenate(groups, axis=0)


_t1 = pl.pallas_call(
    _t1_body,
    grid=(_T1_GRID,),
    in_specs=[pl.BlockSpec((_D, _T1_TOK), lambda g: (0, g))],
    out_specs=pl.BlockSpec((_T1_TOK // 4, 128), lambda g: (g, 0)),
    out_shape=jax.ShapeDtypeStruct((_VP * _D // 128, 128), jnp.float32),
)


def _t3_body(x_ref, y_ref):
    x = x_ref[...].reshape(256, 50, 128)
    parts = [x[:, m, :].T for m in range(50)]
    y_ref[...] = jnp.concatenate(parts, axis=0)


_t3 = pl.pallas_call(
    _t3_body,
    grid=(16,),
    in_specs=[pl.BlockSpec((12800, 128), lambda i: (i, 0))],
    out_specs=pl.BlockSpec((_L * _D, 256), lambda i: (0, i)),
    out_shape=jax.ShapeDtypeStruct((_L * _D, _B), jnp.float32),
)


def _make_emb_kernel():
    mesh = plsc.VectorSubcoreMesh(core_axis_name="c", subcore_axis_name="s")

    @functools.partial(
        pl.kernel,
        mesh=mesh,
        out_type=jax.ShapeDtypeStruct((_N, _D), jnp.float32),
        scratch_types=[
            pltpu.VMEM((_IPW,), jnp.int32),
            pltpu.VMEM((_C, _D), jnp.float32),
            pltpu.VMEM((_C, _D), jnp.float32),
            pltpu.SemaphoreType.DMA,
            pltpu.SemaphoreType.DMA,
            pltpu.SemaphoreType.DMA,
            pltpu.SemaphoreType.DMA,
        ],
        compiler_params=pltpu.CompilerParams(use_tc_tiling_on_sc=False),
    )
    def emb(idx_hbm, table_hbm, out_hbm, idx_v, rows0, rows1,
            gsem0, gsem1, ssem0, ssem1):
        wid = lax.axis_index("c") * 16 + lax.axis_index("s")
        base = wid * _IPW
        pltpu.sync_copy(idx_hbm.at[pl.ds(base, _IPW)], idx_v)

        def xform(i, carry):
            v = idx_v[pl.ds(i * 16, 16)]
            idx_v[pl.ds(i * 16, 16)] = (
                (v & -2048) | ((v & 511) << 2) | ((v & 2047) >> 9))
            return carry

        lax.fori_loop(0, _IPW // 16, xform, 0)

        rows = (rows0, rows1)
        gsem = (gsem0, gsem1)
        ssem = (ssem0, ssem1)

        def fire(c, b):
            pltpu.make_async_copy(
                table_hbm.at[idx_v.at[pl.ds(c * _C, _C)]],
                rows[b], gsem[b]).start()

        def drain_gather(b):
            pltpu.make_async_copy(
                table_hbm.at[idx_v.at[pl.ds(0, _C)]],
                rows[b], gsem[b]).wait()

        def store_start(c, b):
            pltpu.make_async_copy(
                rows[b], out_hbm.at[pl.ds(base + c * _C, _C)],
                ssem[b]).start()

        def store_wait(b):
            pltpu.make_async_copy(
                rows[b], out_hbm.at[pl.ds(base, _C)], ssem[b]).wait()

        fire(0, 0)
        fire(1, 1)

        def body(t, carry):
            for b in range(2):
                c = 2 * t + b

                drain_gather(b)
                store_start(c, b)

                @pl.when(t < _NOUT - 1)
                def _():
                    store_wait(b)
                    fire(c + 2, b)

            return carry

        lax.fori_loop(0, _NOUT, body, 0)
        store_wait(0)
        store_wait(1)

    return emb


_emb = _make_emb_kernel()


@jax.jit
def kernel(token_id_flat, lengths, table):
    idx = token_id_flat.astype(jnp.int32)
    tbl_rm = _t1(table.T).reshape(_VP, _D)
    out = _emb(idx, tbl_rm)
    o_t = _t3(out.reshape(_N * _D // 128, 128))
    return jnp.transpose(o_t.reshape(_L, _D, _B), (2, 0, 1))


# speedup vs baseline: 1.6898x; 1.0209x over previous
import functools

import jax
import jax.numpy as jnp
from jax import lax
from jax.experimental import pallas as pl
from jax.experimental.pallas import tpu as pltpu
from jax.experimental.pallas import tpu_sc as plsc

_B = 4096
_L = 200
_D = 32
_V = 1000000
_N = _B * _L
_NW = 32
_IPW = _N // _NW
_C = 1280
_NCH = _IPW // _C
_NOUT = _NCH // 2

_T1_TOK = 32768
_T1_GRID = -(-_V // _T1_TOK)
_VP = _T1_GRID * _T1_TOK


def _t1_body(x_ref, y_ref):
    x = x_ref[...]
    for k in range(16):
        parts = [x[:, 2048 * k + 512 * j:2048 * k + 512 * (j + 1)].T
                 for j in range(4)]
        y_ref[pl.ds(512 * k, 512), :] = jnp.concatenate(parts, axis=1)


_t1 = pl.pallas_call(
    _t1_body,
    grid=(_T1_GRID,),
    in_specs=[pl.BlockSpec((_D, _T1_TOK), lambda g: (0, g))],
    out_specs=pl.BlockSpec((_T1_TOK // 4, 128), lambda g: (g, 0)),
    out_shape=jax.ShapeDtypeStruct((_VP * _D // 128, 128), jnp.float32),
)


def _t3_body(x_ref, y_ref):
    x = x_ref[...].reshape(256, 50, 128)
    for m in range(50):
        y_ref[pl.ds(128 * m, 128), :] = x[:, m, :].T


_t3 = pl.pallas_call(
    _t3_body,
    grid=(16,),
    in_specs=[pl.BlockSpec((12800, 128), lambda i: (i, 0))],
    out_specs=pl.BlockSpec((_L * _D, 256), lambda i: (0, i)),
    out_shape=jax.ShapeDtypeStruct((_L * _D, _B), jnp.float32),
)


def _make_emb_kernel():
    mesh = plsc.VectorSubcoreMesh(core_axis_name="c", subcore_axis_name="s")

    @functools.partial(
        pl.kernel,
        mesh=mesh,
        out_type=jax.ShapeDtypeStruct((_N, _D), jnp.float32),
        scratch_types=[
            pltpu.VMEM((_IPW,), jnp.int32),
            pltpu.VMEM((_C, _D), jnp.float32),
            pltpu.VMEM((_C, _D), jnp.float32),
            pltpu.SemaphoreType.DMA,
            pltpu.SemaphoreType.DMA,
            pltpu.SemaphoreType.DMA,
            pltpu.SemaphoreType.DMA,
        ],
        compiler_params=pltpu.CompilerParams(use_tc_tiling_on_sc=False),
    )
    def emb(idx_hbm, table_hbm, out_hbm, idx_v, rows0, rows1,
            gsem0, gsem1, ssem0, ssem1):
        wid = lax.axis_index("c") * 16 + lax.axis_index("s")
        base = wid * _IPW
        pltpu.sync_copy(idx_hbm.at[pl.ds(base, _IPW)], idx_v)

        def xform(i, carry):
            for u in range(4):
                v = idx_v[pl.ds(i * 64 + u * 16, 16)]
                idx_v[pl.ds(i * 64 + u * 16, 16)] = (
                    (v & -2048) | ((v & 511) << 2) | ((v & 2047) >> 9))
            return carry

        lax.fori_loop(0, _IPW // 64, xform, 0)

        rows = (rows0, rows1)
        gsem = (gsem0, gsem1)
        ssem = (ssem0, ssem1)

        def fire(c, b):
            pltpu.make_async_copy(
                table_hbm.at[idx_v.at[pl.ds(c * _C, _C)]],
                rows[b], gsem[b]).start()

        def drain_gather(b):
            pltpu.make_async_copy(
                table_hbm.at[idx_v.at[pl.ds(0, _C)]],
                rows[b], gsem[b]).wait()

        def store_start(c, b):
            pltpu.make_async_copy(
                rows[b], out_hbm.at[pl.ds(base + c * _C, _C)],
                ssem[b]).start()

        def store_wait(b):
            pltpu.make_async_copy(
                rows[b], out_hbm.at[pl.ds(base, _C)], ssem[b]).wait()

        fire(0, 0)
        fire(1, 1)

        def body(t, carry):
            for b in range(2):
                c = 2 * t + b

                drain_gather(b)
                store_start(c, b)

                @pl.when(t < _NOUT - 1)
                def _():
                    store_wait(b)
                    fire(c + 2, b)

            return carry

        lax.fori_loop(0, _NOUT, body, 0)
        store_wait(0)
        store_wait(1)

    return emb


_emb = _make_emb_kernel()


@jax.jit
def kernel(token_id_flat, lengths, table):
    idx = token_id_flat.astype(jnp.int32)
    tbl_rm = _t1(table.T).reshape(_VP, _D)
    out = _emb(idx, tbl_rm)
    o_t = _t3(out.reshape(_N * _D // 128, 128))
    return jnp.transpose(o_t.reshape(_L, _D, _B), (2, 0, 1))
